# baseline (speedup 1.0000x reference)
import jax
import jax.numpy as jnp
from jax.experimental import pallas as pl
from jax.experimental.pallas import tpu as pltpu

_N = 16384
_D = 4096
_E = 64
_K = 8
_T = 512
_Z_LOSS_W = 0.001
_AUX_W = 0.01


def _router_body(x_ref, w_ref, idx_ref, rw_ref, z_ref, p_ref):
    x = x_ref[...]
    w = w_ref[...]
    lt = jax.lax.dot_general(w, x, (((1,), (1,)), ((), ())),
                             preferred_element_type=jnp.float32)

    m = jnp.max(lt, axis=0, keepdims=True)
    e = jnp.exp(lt - m)
    s = jnp.sum(e, axis=0, keepdims=True)
    lse = m + jnp.log(s)
    z_ref[...] = jnp.sum(lse * lse, axis=(0, 1), keepdims=True).reshape(1, 1, 1)
    p_ref[...] = jnp.sum(e / s, axis=1, keepdims=True).reshape(1, 1, _E)

    iota = jax.lax.broadcasted_iota(jnp.int32, lt.shape, 0)
    l = lt
    vals, idxs = [], []
    for _ in range(_K):
        mk = jnp.max(l, axis=0, keepdims=True)
        ik = jnp.min(jnp.where(l == mk, iota, _E), axis=0, keepdims=True)
        vals.append(mk)
        idxs.append(ik)
        l = jnp.where(iota == ik, -jnp.inf, l)
    v = jnp.concatenate(vals, axis=0)
    ii = jnp.concatenate(idxs, axis=0)
    ev = jnp.exp(v - v[:1, :])
    rw = ev / jnp.sum(ev, axis=0, keepdims=True)
    rw_ref[...] = rw.T
    idx_ref[...] = ii.T


def kernel(x, W):
    grid = _N // _T
    idx, rw, zp, pp = pl.pallas_call(
        _router_body,
        grid=(grid,),
        in_specs=[
            pl.BlockSpec((_T, _D), lambda i: (i, 0)),
            pl.BlockSpec((_E, _D), lambda i: (0, 0)),
        ],
        out_specs=[
            pl.BlockSpec((_T, _K), lambda i: (i, 0)),
            pl.BlockSpec((_T, _K), lambda i: (i, 0)),
            pl.BlockSpec((1, 1, 1), lambda i: (i, 0, 0)),
            pl.BlockSpec((1, 1, _E), lambda i: (i, 0, 0)),
        ],
        out_shape=[
            jax.ShapeDtypeStruct((_N, _K), jnp.int32),
            jax.ShapeDtypeStruct((_N, _K), jnp.float32),
            jax.ShapeDtypeStruct((grid, 1, 1), jnp.float32),
            jax.ShapeDtypeStruct((grid, 1, _E), jnp.float32),
        ],
        compiler_params=pltpu.CompilerParams(
            dimension_semantics=("parallel",),
        ),
    )(x, W)
    z_loss = (jnp.sum(zp) / _N) * _Z_LOSS_W
    lb_loss = _AUX_W * _AUX_W * (jnp.sum(pp) / _N - float(_K))
    return idx, rw, (z_loss + lb_loss).astype(jnp.float32)


# speedup vs baseline: 1.0709x; 1.0709x over previous
import jax
import jax.numpy as jnp
from jax.experimental import pallas as pl
from jax.experimental.pallas import tpu as pltpu

_N = 16384
_D = 4096
_E = 64
_K = 8
_T = 1024
_Z_LOSS_W = 0.001
_AUX_W = 0.01


def _router_body(x_ref, w_ref, idx_ref, rw_ref, z_ref, p_ref):
    x = x_ref[...]
    w = w_ref[...]
    lt = jax.lax.dot_general(w, x, (((1,), (1,)), ((), ())),
                             preferred_element_type=jnp.float32)

    m = jnp.max(lt, axis=0, keepdims=True)
    e = jnp.exp(lt - m)
    s = jnp.sum(e, axis=0, keepdims=True)
    lse = m + jnp.log(s)
    z_ref[...] = jnp.sum(lse * lse, axis=(0, 1), keepdims=True).reshape(1, 1, 1)
    p_ref[...] = jnp.sum(e / s, axis=1, keepdims=True).reshape(1, 1, _E)

    iota = jax.lax.broadcasted_iota(jnp.int32, lt.shape, 0)
    l = lt
    vals, idxs = [], []
    for _ in range(_K):
        mk = jnp.max(l, axis=0, keepdims=True)
        ik = jnp.min(jnp.where(l == mk, iota, _E), axis=0, keepdims=True)
        vals.append(mk)
        idxs.append(ik)
        l = jnp.where(iota == ik, -jnp.inf, l)
    v = jnp.concatenate(vals, axis=0)
    ii = jnp.concatenate(idxs, axis=0)
    ev = jnp.exp(v - v[:1, :])
    rw = ev / jnp.sum(ev, axis=0, keepdims=True)
    rw_ref[...] = rw.T
    idx_ref[...] = ii.T


def kernel(x, W):
    grid = _N // _T
    idx, rw, zp, pp = pl.pallas_call(
        _router_body,
        grid=(grid,),
        in_specs=[
            pl.BlockSpec((_T, _D), lambda i: (i, 0)),
            pl.BlockSpec((_E, _D), lambda i: (0, 0)),
        ],
        out_specs=[
            pl.BlockSpec((_T, _K), lambda i: (i, 0)),
            pl.BlockSpec((_T, _K), lambda i: (i, 0)),
            pl.BlockSpec((1, 1, 1), lambda i: (i, 0, 0)),
            pl.BlockSpec((1, 1, _E), lambda i: (i, 0, 0)),
        ],
        out_shape=[
            jax.ShapeDtypeStruct((_N, _K), jnp.int32),
            jax.ShapeDtypeStruct((_N, _K), jnp.float32),
            jax.ShapeDtypeStruct((grid, 1, 1), jnp.float32),
            jax.ShapeDtypeStruct((grid, 1, _E), jnp.float32),
        ],
        compiler_params=pltpu.CompilerParams(
            dimension_semantics=("parallel",),
        ),
    )(x, W)
    z_loss = (jnp.sum(zp) / _N) * _Z_LOSS_W
    lb_loss = _AUX_W * _AUX_W * (jnp.sum(pp) / _N - float(_K))
    return idx, rw, (z_loss + lb_loss).astype(jnp.float32)
